# baseline (speedup 1.0000x reference)
import functools

import jax
import jax.numpy as jnp
from jax import lax
from jax.experimental import pallas as pl
from jax.experimental.pallas import tpu as pltpu
from jax.experimental.pallas import tpu_sc as plsc

B, LIG, PROT, PN, H, E = 4096, 384, 1536, 8, 256, 8
BLK = 256
NB = B // BLK
NEG_SLOPE = 0.01


def _leaky(v):
    return jnp.maximum(v, NEG_SLOPE * v)


def _dot(a, b):
    return jnp.dot(a, b, preferred_element_type=jnp.float32)


def _dot_t(a, b):
    return lax.dot_general(a, b, (((0,), (1,)), ((), ())),
                           preferred_element_type=jnp.float32)


def _tc_body(lig_ref, prot_ref, Wl_ref, bl_ref, Wp_ref, bp_ref, Wr_ref,
             W1_ref, b1_ref, W2_ref, b2_ref, logits_ref, vals_ref):
    x = _dot(lig_ref[:], Wl_ref[:]) + bl_ref[:]
    for n in range(PN):
        p = _dot(prot_ref[:, n, :], Wp_ref[n]) + bp_ref[n]
        x += _leaky(p)
    logits_ref[:] = _dot_t(Wr_ref[:], x)
    rows = []
    for e in range(E):
        hT = _leaky(_dot_t(W1_ref[e], x) + b1_ref[e])
        rows.append(lax.dot_general(W2_ref[e], hT, (((0,), (0,)), ((), ())),
                                    preferred_element_type=jnp.float32))
    vals_ref[:] = jnp.concatenate(rows, axis=0) + b2_ref[:]


@functools.partial(jax.jit, static_argnames=())
def _tc_stage(lig_x, prot_x, W_lig, b_lig, W_prot, b_prot, W_router, W1, b1,
              W2r, b2r):
    return pl.pallas_call(
        _tc_body,
        grid=(NB,),
        in_specs=[
            pl.BlockSpec((BLK, LIG), lambda i: (i, 0)),
            pl.BlockSpec((BLK, PN, PROT), lambda i: (i, 0, 0)),
            pl.BlockSpec((LIG, H), lambda i: (0, 0)),
            pl.BlockSpec((1, H), lambda i: (0, 0)),
            pl.BlockSpec((PN, PROT, H), lambda i: (0, 0, 0)),
            pl.BlockSpec((PN, 1, H), lambda i: (0, 0, 0)),
            pl.BlockSpec((H, E), lambda i: (0, 0)),
            pl.BlockSpec((E, H, H), lambda i: (0, 0, 0)),
            pl.BlockSpec((E, H, 1), lambda i: (0, 0, 0)),
            pl.BlockSpec((E, H, 1), lambda i: (0, 0, 0)),
            pl.BlockSpec((E, 1), lambda i: (0, 0)),
        ],
        out_specs=[
            pl.BlockSpec((E, BLK), lambda i: (0, i)),
            pl.BlockSpec((E, BLK), lambda i: (0, i)),
        ],
        out_shape=[
            jax.ShapeDtypeStruct((E, B), jnp.float32),
            jax.ShapeDtypeStruct((E, B), jnp.float32),
        ],
        compiler_params=pltpu.CompilerParams(
            dimension_semantics=("arbitrary",)),
    )(lig_x, prot_x, W_lig, b_lig, W_prot, b_prot, W_router, W1, b1, W2r, b2r)


def _make_sc_route():
    info = plsc.get_sparse_core_info()
    NC, NS, L = info.num_cores, info.num_subcores, info.num_lanes
    NW = NC * NS
    bw = B // NW
    nch = bw // L
    mesh = plsc.VectorSubcoreMesh(core_axis_name="c", subcore_axis_name="s")
    BIG = jnp.float32(3.0e38)

    @functools.partial(
        pl.kernel, mesh=mesh,
        out_type=jax.ShapeDtypeStruct((B,), jnp.float32),
        scratch_types=[
            pltpu.VMEM((E, bw), jnp.float32),
            pltpu.VMEM((E, bw), jnp.float32),
            pltpu.VMEM((bw,), jnp.float32),
        ],
    )
    def route(logitsT_hbm, valsT_hbm, out_hbm, lg_v, vl_v, out_v):
        wid = lax.axis_index("s") * NC + lax.axis_index("c")
        base = wid * bw
        pltpu.sync_copy(logitsT_hbm.at[:, pl.ds(base, bw)], lg_v)
        pltpu.sync_copy(valsT_hbm.at[:, pl.ds(base, bw)], vl_v)
        for c in range(nch):
            t0 = c * L
            l = [lg_v[e, pl.ds(t0, L)] for e in range(E)]
            m1 = l[0]
            for e in range(1, E):
                m1 = jnp.maximum(m1, l[e])
            i1 = jnp.zeros((L,), jnp.int32)
            for e in range(E - 1, -1, -1):
                i1 = jnp.where(l[e] == m1, jnp.full((L,), e, jnp.int32), i1)
            l2 = [jnp.where(i1 == jnp.full((L,), e, jnp.int32),
                            jnp.full((L,), -BIG, jnp.float32), l[e])
                  for e in range(E)]
            m2 = l2[0]
            for e in range(1, E):
                m2 = jnp.maximum(m2, l2[e])
            i2 = jnp.zeros((L,), jnp.int32)
            for e in range(E - 1, -1, -1):
                i2 = jnp.where(l2[e] == m2, jnp.full((L,), e, jnp.int32), i2)
            v1 = jnp.zeros((L,), jnp.float32)
            v2 = jnp.zeros((L,), jnp.float32)
            for e in range(E):
                ve = vl_v[e, pl.ds(t0, L)]
                sel_e = jnp.full((L,), e, jnp.int32)
                v1 = jnp.where(i1 == sel_e, ve, v1)
                v2 = jnp.where(i2 == sel_e, ve, v2)
            t = jnp.exp(m2 - m1)
            w1 = 1.0 / (1.0 + t)
            out_v[pl.ds(t0, L)] = w1 * v1 + (1.0 - w1) * v2
        pltpu.sync_copy(out_v, out_hbm.at[pl.ds(base, bw)])

    return route


_sc_route = None


def kernel(lig_x, prot_x, W_lig, b_lig, W_prot, b_prot, W_router, W1, b1, W2, b2):
    global _sc_route
    if _sc_route is None:
        _sc_route = _make_sc_route()
    logitsT, valsT = _tc_stage(
        lig_x, prot_x, W_lig, b_lig.reshape(1, H), W_prot,
        b_prot.reshape(PN, 1, H), W_router, W1, b1.reshape(E, H, 1),
        W2, b2)
    out = _sc_route(logitsT, valsT)
    return out[:, None]


# speedup vs baseline: 1.0851x; 1.0851x over previous
import functools

import jax
import jax.numpy as jnp
from jax import lax
from jax.experimental import pallas as pl
from jax.experimental.pallas import tpu as pltpu
from jax.experimental.pallas import tpu_sc as plsc

B, LIG, PROT, PN, H, E = 4096, 384, 1536, 8, 256, 8
BLK = 256
NB = B // BLK
NEG_SLOPE = 0.01


def _leaky(v):
    return jnp.maximum(v, NEG_SLOPE * v)


def _dot(a, b):
    return jnp.dot(a, b, preferred_element_type=jnp.float32)


def _dot_t(a, b):
    return lax.dot_general(a, b, (((0,), (1,)), ((), ())),
                           preferred_element_type=jnp.float32)


def _tc_body(lig_ref, prot_ref, Wl_ref, bl_ref, Wp_ref, bp_ref, Wr_ref,
             W1_ref, b1_ref, W2_ref, b2_ref, logits_ref, vals_ref):
    x = _dot(lig_ref[:], Wl_ref[:]) + bl_ref[:]
    for n in range(PN):
        p = _dot(prot_ref[:, n, :], Wp_ref[n]) + bp_ref[n]
        x += _leaky(p)
    logits_ref[:] = _dot_t(Wr_ref[:], x)
    rows = []
    for e in range(E):
        hT = _leaky(_dot_t(W1_ref[e], x) + b1_ref[e])
        hb = hT.astype(jnp.bfloat16).astype(jnp.float32)
        wb = W2_ref[e].astype(jnp.bfloat16).astype(jnp.float32)
        rows.append(jnp.sum(hb * wb, axis=0).reshape(1, BLK))
    vals_ref[:] = jnp.concatenate(rows, axis=0) + b2_ref[:]


@functools.partial(jax.jit, static_argnames=())
def _tc_stage(lig_x, prot_x, W_lig, b_lig, W_prot, b_prot, W_router, W1, b1,
              W2r, b2r):
    return pl.pallas_call(
        _tc_body,
        grid=(NB,),
        in_specs=[
            pl.BlockSpec((BLK, LIG), lambda i: (i, 0)),
            pl.BlockSpec((BLK, PN, PROT), lambda i: (i, 0, 0)),
            pl.BlockSpec((LIG, H), lambda i: (0, 0)),
            pl.BlockSpec((1, H), lambda i: (0, 0)),
            pl.BlockSpec((PN, PROT, H), lambda i: (0, 0, 0)),
            pl.BlockSpec((PN, 1, H), lambda i: (0, 0, 0)),
            pl.BlockSpec((H, E), lambda i: (0, 0)),
            pl.BlockSpec((E, H, H), lambda i: (0, 0, 0)),
            pl.BlockSpec((E, H, 1), lambda i: (0, 0, 0)),
            pl.BlockSpec((E, H, 1), lambda i: (0, 0, 0)),
            pl.BlockSpec((E, 1), lambda i: (0, 0)),
        ],
        out_specs=[
            pl.BlockSpec((E, BLK), lambda i: (0, i)),
            pl.BlockSpec((E, BLK), lambda i: (0, i)),
        ],
        out_shape=[
            jax.ShapeDtypeStruct((E, B), jnp.float32),
            jax.ShapeDtypeStruct((E, B), jnp.float32),
        ],
        compiler_params=pltpu.CompilerParams(
            dimension_semantics=("arbitrary",)),
    )(lig_x, prot_x, W_lig, b_lig, W_prot, b_prot, W_router, W1, b1, W2r, b2r)


def _make_sc_route():
    info = plsc.get_sparse_core_info()
    NC, NS, L = info.num_cores, info.num_subcores, info.num_lanes
    NW = NC * NS
    bw = B // NW
    nch = bw // L
    mesh = plsc.VectorSubcoreMesh(core_axis_name="c", subcore_axis_name="s")
    BIG = jnp.float32(3.0e38)

    @functools.partial(
        pl.kernel, mesh=mesh,
        out_type=jax.ShapeDtypeStruct((B,), jnp.float32),
        scratch_types=[
            pltpu.VMEM((E, bw), jnp.float32),
            pltpu.VMEM((E, bw), jnp.float32),
            pltpu.VMEM((bw,), jnp.float32),
        ],
    )
    def route(logitsT_hbm, valsT_hbm, out_hbm, lg_v, vl_v, out_v):
        wid = lax.axis_index("s") * NC + lax.axis_index("c")
        base = wid * bw
        pltpu.sync_copy(logitsT_hbm.at[:, pl.ds(base, bw)], lg_v)
        pltpu.sync_copy(valsT_hbm.at[:, pl.ds(base, bw)], vl_v)
        for c in range(nch):
            t0 = c * L
            l = [lg_v[e, pl.ds(t0, L)] for e in range(E)]
            m1 = l[0]
            for e in range(1, E):
                m1 = jnp.maximum(m1, l[e])
            i1 = jnp.zeros((L,), jnp.int32)
            for e in range(E - 1, -1, -1):
                i1 = jnp.where(l[e] == m1, jnp.full((L,), e, jnp.int32), i1)
            l2 = [jnp.where(i1 == jnp.full((L,), e, jnp.int32),
                            jnp.full((L,), -BIG, jnp.float32), l[e])
                  for e in range(E)]
            m2 = l2[0]
            for e in range(1, E):
                m2 = jnp.maximum(m2, l2[e])
            i2 = jnp.zeros((L,), jnp.int32)
            for e in range(E - 1, -1, -1):
                i2 = jnp.where(l2[e] == m2, jnp.full((L,), e, jnp.int32), i2)
            v1 = jnp.zeros((L,), jnp.float32)
            v2 = jnp.zeros((L,), jnp.float32)
            for e in range(E):
                ve = vl_v[e, pl.ds(t0, L)]
                sel_e = jnp.full((L,), e, jnp.int32)
                v1 = jnp.where(i1 == sel_e, ve, v1)
                v2 = jnp.where(i2 == sel_e, ve, v2)
            t = jnp.exp(m2 - m1)
            w1 = 1.0 / (1.0 + t)
            out_v[pl.ds(t0, L)] = w1 * v1 + (1.0 - w1) * v2
        pltpu.sync_copy(out_v, out_hbm.at[pl.ds(base, bw)])

    return route


_sc_route = None


def kernel(lig_x, prot_x, W_lig, b_lig, W_prot, b_prot, W_router, W1, b1, W2, b2):
    global _sc_route
    if _sc_route is None:
        _sc_route = _make_sc_route()
    logitsT, valsT = _tc_stage(
        lig_x, prot_x, W_lig, b_lig.reshape(1, H), W_prot,
        b_prot.reshape(PN, 1, H), W_router, W1, b1.reshape(E, H, 1),
        W2, b2)
    out = _sc_route(logitsT, valsT)
    return out[:, None]
